# Initial kernel scaffold; baseline (speedup 1.0000x reference)
import jax
import jax.numpy as jnp
from jax.experimental import pallas as pl


def kernel(x_protein, x_disease, edge_index_pp, edge_attr_pp, edge_index_pd, sup_edge_index, W_self_p, W_nbr_pp, W_self_d, W_nbr_pd):
    raise NotImplementedError("write your pallas kernel here")


# speedup vs baseline: 2.8985x; 2.8985x over previous
import dataclasses
import functools

import jax
import jax.numpy as jnp
from jax import lax
from jax.experimental import pallas as pl
from jax.experimental.pallas import tpu as pltpu
from jax.experimental.pallas import tpu_sc as plsc

N_PROT = 10000
N_DIS = 10000
E_PP = 320000
E_PD = 160000
E_SUP = 100000
D = 128

NC = 2
NS = 16
NW = NC * NS
CH = 80
RB = 80

_mesh = plsc.VectorSubcoreMesh(core_axis_name="c", subcore_axis_name="s",
                               num_cores=NC, num_subcores=NS)

_sc_params = pltpu.CompilerParams()
if "needs_layout_passes" in pltpu.CompilerParams.__dataclass_fields__:
    _sc_params = dataclasses.replace(_sc_params, needs_layout_passes=False)


def _fill_zeros(zrow_v):
    z16 = jnp.zeros((16,), jnp.float32)

    @pl.loop(0, RB)
    def _(i):
        for j in range(D // 16):
            zrow_v[i, pl.ds(j * 16, 16)] = z16


def _zero_acc(zrow_v, acc_sh, s):
    @pl.loop(s, N_PROT // RB, step=NS)
    def _(t):
        pltpu.sync_copy(zrow_v, acc_sh.at[pl.ds(t * RB, RB)])


def _write_acc(acc_sh, out_hbm, c, s):
    @pl.loop(s, N_PROT // RB, step=NS)
    def _(t):
        pltpu.sync_copy(acc_sh.at[pl.ds(t * RB, RB)],
                        out_hbm.at[c].at[pl.ds(t * RB, RB)])


def _agg_body(xp_hbm, srcpp_hbm, dstpp_hbm, w_hbm, srcpd_hbm, dstpd_hbm,
              aggpp_hbm, aggpd_hbm,
              idx_v, dst_v, w_v, rows_v, zrow_v, acc_sh, sem):
    c = lax.axis_index("c")
    s = lax.axis_index("s")
    wid = s * NC + c

    _fill_zeros(zrow_v)
    _zero_acc(zrow_v, acc_sh, s)
    plsc.subcore_barrier()

    @pl.loop(wid, E_PP // CH, step=NW)
    def _(k):
        base = k * CH
        pltpu.sync_copy(srcpp_hbm.at[pl.ds(base, CH)], idx_v)
        pltpu.sync_copy(dstpp_hbm.at[pl.ds(base, CH)], dst_v)
        pltpu.sync_copy(w_hbm.at[pl.ds(base, CH)], w_v)
        pltpu.async_copy(xp_hbm.at[idx_v], rows_v, sem).wait()

        @pl.loop(0, CH)
        def _(e):
            wv = plsc.load_gather(w_v, [jnp.full((16,), e, jnp.int32)])
            for j in range(D // 16):
                sl = pl.ds(j * 16, 16)
                rows_v[e, sl] = rows_v[e, sl] * wv

        pltpu.sync_copy(rows_v, acc_sh.at[dst_v], add=True)

    plsc.subcore_barrier()
    _write_acc(acc_sh, aggpp_hbm, c, s)
    _zero_acc(zrow_v, acc_sh, s)
    plsc.subcore_barrier()

    @pl.loop(wid, E_PD // CH, step=NW)
    def _(k):
        base = k * CH
        pltpu.sync_copy(srcpd_hbm.at[pl.ds(base, CH)], idx_v)
        pltpu.sync_copy(dstpd_hbm.at[pl.ds(base, CH)], dst_v)
        pltpu.async_copy(xp_hbm.at[idx_v], rows_v, sem).wait()
        pltpu.sync_copy(rows_v, acc_sh.at[dst_v], add=True)

    plsc.subcore_barrier()
    _write_acc(acc_sh, aggpd_hbm, c, s)


@jax.jit
def _aggregate(xp, src_pp, dst_pp, w_pp, src_pd, dst_pd):
    f = pl.kernel(
        _agg_body,
        out_type=(jax.ShapeDtypeStruct((NC, N_PROT, D), jnp.float32),
                  jax.ShapeDtypeStruct((NC, N_DIS, D), jnp.float32)),
        mesh=_mesh,
        scratch_types=[
            pltpu.VMEM((CH,), jnp.int32),
            pltpu.VMEM((CH,), jnp.int32),
            pltpu.VMEM((CH,), jnp.float32),
            pltpu.VMEM((CH, D), jnp.float32),
            pltpu.VMEM((RB, D), jnp.float32),
            pltpu.VMEM_SHARED((N_PROT, D), jnp.float32),
            pltpu.SemaphoreType.DMA,
        ],
        compiler_params=_sc_params,
    )
    return f(xp, src_pp, dst_pp, w_pp, src_pd, dst_pd)


def _enc_block(x_ref, a_ref, ws_ref, wn_ref, o_ref):
    agg = a_ref[0] + a_ref[1]
    o_ref[...] = jnp.maximum(
        jnp.dot(x_ref[...], ws_ref[...], preferred_element_type=jnp.float32)
        + jnp.dot(agg, wn_ref[...], preferred_element_type=jnp.float32),
        0.0)


@jax.jit
def _encode(x, agg2, w_self, w_nbr):
    n = x.shape[0]
    br = 2000
    return pl.pallas_call(
        _enc_block,
        grid=(n // br,),
        in_specs=[
            pl.BlockSpec((br, D), lambda i: (i, 0)),
            pl.BlockSpec((NC, br, D), lambda i: (0, i, 0)),
            pl.BlockSpec((D, D), lambda i: (0, 0)),
            pl.BlockSpec((D, D), lambda i: (0, 0)),
        ],
        out_specs=pl.BlockSpec((br, D), lambda i: (i, 0)),
        out_shape=jax.ShapeDtypeStruct((n, D), jnp.float32),
    )(x, agg2, w_self, w_nbr)


def _decode_body(hp_hbm, hd_hbm, row_hbm, col_hbm, out_hbm,
                 ridx_v, cidx_v, l_v, r_v, o_v, sem):
    c = lax.axis_index("c")
    s = lax.axis_index("s")
    wid = s * NC + c
    lane = lax.iota(jnp.int32, 16)

    @pl.loop(wid, E_SUP // CH, step=NW)
    def _(k):
        base = k * CH
        pltpu.sync_copy(row_hbm.at[pl.ds(base, CH)], ridx_v)
        pltpu.sync_copy(col_hbm.at[pl.ds(base, CH)], cidx_v)
        pltpu.async_copy(hp_hbm.at[ridx_v], l_v, sem).wait()
        pltpu.async_copy(hd_hbm.at[cidx_v], r_v, sem).wait()

        @pl.loop(0, CH // 16)
        def _(g):
            out16 = jnp.zeros((16,), jnp.float32)
            for r in range(16):
                e = g * 16 + r
                acc = l_v[e, pl.ds(0, 16)] * r_v[e, pl.ds(0, 16)]
                for j in range(1, D // 16):
                    sl = pl.ds(j * 16, 16)
                    acc = acc + l_v[e, sl] * r_v[e, sl]
                dot = jnp.sum(acc)
                out16 = jnp.where(lane == r, dot, out16)
            o_v[pl.ds(g * 16, 16)] = out16

        pltpu.sync_copy(o_v, out_hbm.at[pl.ds(base, CH)])


@jax.jit
def _decode(hp, hd, row, col):
    f = pl.kernel(
        _decode_body,
        out_type=jax.ShapeDtypeStruct((E_SUP,), jnp.float32),
        mesh=_mesh,
        scratch_types=[
            pltpu.VMEM((CH,), jnp.int32),
            pltpu.VMEM((CH,), jnp.int32),
            pltpu.VMEM((CH, D), jnp.float32),
            pltpu.VMEM((CH, D), jnp.float32),
            pltpu.VMEM((CH,), jnp.float32),
            pltpu.SemaphoreType.DMA,
        ],
        compiler_params=_sc_params,
    )
    return f(hp, hd, row, col)


def kernel(x_protein, x_disease, edge_index_pp, edge_attr_pp, edge_index_pd,
           sup_edge_index, W_self_p, W_nbr_pp, W_self_d, W_nbr_pd):
    w_pp = edge_attr_pp[:, 0]
    src_pp, dst_pp = edge_index_pp[0], edge_index_pp[1]
    src_pd, dst_pd = edge_index_pd[0], edge_index_pd[1]
    row, col = sup_edge_index[0], sup_edge_index[1]

    aggpp2, aggpd2 = _aggregate(x_protein, src_pp, dst_pp, w_pp,
                                src_pd, dst_pd)
    h_protein = _encode(x_protein, aggpp2, W_self_p, W_nbr_pp)
    h_disease = _encode(x_disease, aggpd2, W_self_d, W_nbr_pd)
    return _decode(h_protein, h_disease, row, col)
